# initial kernel scaffold (unmeasured)
import functools

import jax
import jax.numpy as jnp
from jax import lax
from jax.experimental import pallas as pl
from jax.experimental.pallas import tpu as pltpu

T = 4096
T_HALF = T // 2
V_SHARD = 8192
D = 2048
DMA_WINDOW = 64


def kernel(ids, E):
    def body(rc_ref, mask_ref, e_ref, out_ref,
             gbuf, ysend, yrecv, xrecv,
             gsem, osem, send_sems, recv_sems):
        my_x = lax.axis_index("x")
        my_y = lax.axis_index("y")

        barrier_sem = pltpu.get_barrier_semaphore()
        pl.semaphore_signal(barrier_sem, inc=1, device_id=(my_x, 1 - my_y),
                            device_id_type=pl.DeviceIdType.MESH)
        pl.semaphore_signal(barrier_sem, inc=1, device_id=(1 - my_x, my_y),
                            device_id_type=pl.DeviceIdType.MESH)
        pl.semaphore_wait(barrier_sem, 2)

        def row_copy(i):
            return pltpu.make_async_copy(
                e_ref.at[pl.ds(rc_ref[i], 1)],
                gbuf.at[pl.ds(i, 1)],
                gsem,
            )

        def issue(i, _):
            row_copy(i).start()

            @pl.when(i >= DMA_WINDOW)
            def _():
                row_copy(i - DMA_WINDOW).wait()

            return 0

        lax.fori_loop(0, T_HALF, issue, 0)

        def drain(i, _):
            row_copy(i).wait()
            return 0

        lax.fori_loop(T_HALF - DMA_WINDOW, T_HALF, drain, 0)

        ysend[...] = (gbuf[...] * mask_ref[...]).astype(jnp.bfloat16)

        rdma_y = pltpu.make_async_remote_copy(
            src_ref=ysend,
            dst_ref=yrecv,
            send_sem=send_sems.at[0],
            recv_sem=recv_sems.at[0],
            device_id=(my_x, 1 - my_y),
            device_id_type=pl.DeviceIdType.MESH,
        )
        rdma_y.start()
        rdma_y.wait()

        gbuf[...] = ysend[...].astype(jnp.float32) + yrecv[...].astype(jnp.float32)
        own_store = pltpu.make_async_copy(
            gbuf, out_ref.at[pl.ds(my_x * T_HALF, T_HALF)], osem)
        own_store.start()

        ysend[...] = ysend[...] + yrecv[...]

        rdma_x = pltpu.make_async_remote_copy(
            src_ref=ysend,
            dst_ref=xrecv,
            send_sem=send_sems.at[1],
            recv_sem=recv_sems.at[1],
            device_id=(1 - my_x, my_y),
            device_id_type=pl.DeviceIdType.MESH,
        )
        rdma_x.start()
        rdma_x.wait()

        own_store.wait()
        gbuf[...] = xrecv[...].astype(jnp.float32)
        other_store = pltpu.make_async_copy(
            gbuf, out_ref.at[pl.ds((1 - my_x) * T_HALF, T_HALF)], osem)
        other_store.start()
        other_store.wait()

    my_x = lax.axis_index("x")
    my_y = lax.axis_index("y")
    ids_half = lax.dynamic_slice(ids, (my_x * T_HALF,), (T_HALF,))
    r = ids_half - my_y * V_SHARD
    mask = ((r >= 0) & (r < V_SHARD)).astype(jnp.float32).reshape(T_HALF, 1)
    rc = jnp.clip(r, 0, V_SHARD - 1).astype(jnp.int32)

    return pl.pallas_call(
        body,
        out_shape=jax.ShapeDtypeStruct((T, D), jnp.float32),
        in_specs=[
            pl.BlockSpec(memory_space=pltpu.SMEM),
            pl.BlockSpec(memory_space=pltpu.VMEM),
            pl.BlockSpec(memory_space=pltpu.ANY),
        ],
        out_specs=pl.BlockSpec(memory_space=pltpu.ANY),
        scratch_shapes=[
            pltpu.VMEM((T_HALF, D), jnp.float32),
            pltpu.VMEM((T_HALF, D), jnp.bfloat16),
            pltpu.VMEM((T_HALF, D), jnp.bfloat16),
            pltpu.VMEM((T_HALF, D), jnp.bfloat16),
            pltpu.SemaphoreType.DMA,
            pltpu.SemaphoreType.DMA,
            pltpu.SemaphoreType.DMA((2,)),
            pltpu.SemaphoreType.DMA((2,)),
        ],
        compiler_params=pltpu.CompilerParams(collective_id=0),
    )(rc, mask, E)


# baseline (device time: 275856 ns/iter reference)
import functools

import jax
import jax.numpy as jnp
from jax import lax
from jax.experimental import pallas as pl
from jax.experimental.pallas import tpu as pltpu

T = 4096
T_HALF = T // 2
V_SHARD = 8192
D = 2048
G_BLOCK = 512
DMA_WINDOW = 64


def kernel(ids, E):
    def body(rc_ref, mask_ref, e_ref, out_ref,
             gbuf, ysend, yrecv, xrecv,
             gsem, osem, send_sems, recv_sems):
        my_x = lax.axis_index("x")
        my_y = lax.axis_index("y")

        barrier_sem = pltpu.get_barrier_semaphore()
        pl.semaphore_signal(barrier_sem, inc=1, device_id=(my_x, 1 - my_y),
                            device_id_type=pl.DeviceIdType.MESH)
        pl.semaphore_signal(barrier_sem, inc=1, device_id=(1 - my_x, my_y),
                            device_id_type=pl.DeviceIdType.MESH)
        pl.semaphore_wait(barrier_sem, 2)

        def row_copy(i, base):
            return pltpu.make_async_copy(
                e_ref.at[pl.ds(rc_ref[i], 1)],
                gbuf.at[pl.ds(i - base, 1)],
                gsem,
            )

        for k in range(T_HALF // G_BLOCK):
            base = k * G_BLOCK

            def issue(i, _, base=base):
                row_copy(i, base).start()

                @pl.when(i >= base + DMA_WINDOW)
                def _():
                    row_copy(i - DMA_WINDOW, base).wait()

                return 0

            lax.fori_loop(base, base + G_BLOCK, issue, 0)

            def drain(i, _, base=base):
                row_copy(i, base).wait()
                return 0

            lax.fori_loop(base + G_BLOCK - DMA_WINDOW, base + G_BLOCK, drain, 0)

            ysend[pl.ds(base, G_BLOCK)] = (
                gbuf[...] * mask_ref[pl.ds(base, G_BLOCK)]
            ).astype(jnp.bfloat16)

        rdma_y = pltpu.make_async_remote_copy(
            src_ref=ysend,
            dst_ref=yrecv,
            send_sem=send_sems.at[0],
            recv_sem=recv_sems.at[0],
            device_id=(my_x, 1 - my_y),
            device_id_type=pl.DeviceIdType.MESH,
        )
        rdma_y.start()
        rdma_y.wait()

        ysend[...] = ysend[...] + yrecv[...]
        own_store = pltpu.make_async_copy(
            ysend, out_ref.at[pl.ds(my_x * T_HALF, T_HALF)], osem)
        own_store.start()

        rdma_x = pltpu.make_async_remote_copy(
            src_ref=ysend,
            dst_ref=xrecv,
            send_sem=send_sems.at[1],
            recv_sem=recv_sems.at[1],
            device_id=(1 - my_x, my_y),
            device_id_type=pl.DeviceIdType.MESH,
        )
        rdma_x.start()
        rdma_x.wait()

        other_store = pltpu.make_async_copy(
            xrecv, out_ref.at[pl.ds((1 - my_x) * T_HALF, T_HALF)], osem)
        other_store.start()
        own_store.wait()
        other_store.wait()

    my_x = lax.axis_index("x")
    my_y = lax.axis_index("y")
    ids_half = lax.dynamic_slice(ids, (my_x * T_HALF,), (T_HALF,))
    r = ids_half - my_y * V_SHARD
    mask = ((r >= 0) & (r < V_SHARD)).astype(jnp.float32).reshape(T_HALF, 1)
    rc = jnp.clip(r, 0, V_SHARD - 1).astype(jnp.int32)

    return pl.pallas_call(
        body,
        out_shape=jax.ShapeDtypeStruct((T, D), jnp.bfloat16),
        in_specs=[
            pl.BlockSpec(memory_space=pltpu.SMEM),
            pl.BlockSpec(memory_space=pltpu.VMEM),
            pl.BlockSpec(memory_space=pl.ANY),
        ],
        out_specs=pl.BlockSpec(memory_space=pl.ANY),
        scratch_shapes=[
            pltpu.VMEM((G_BLOCK, D), jnp.float32),
            pltpu.VMEM((T_HALF, D), jnp.bfloat16),
            pltpu.VMEM((T_HALF, D), jnp.bfloat16),
            pltpu.VMEM((T_HALF, D), jnp.bfloat16),
            pltpu.SemaphoreType.DMA,
            pltpu.SemaphoreType.DMA,
            pltpu.SemaphoreType.DMA((2,)),
            pltpu.SemaphoreType.DMA((2,)),
        ],
        compiler_params=pltpu.CompilerParams(collective_id=0),
    )(rc, mask, E)


# device time: 181158 ns/iter; 1.5227x vs baseline; 1.5227x over previous
import jax
import jax.numpy as jnp
from jax import lax
from jax.experimental import pallas as pl
from jax.experimental.pallas import tpu as pltpu

T = 4096
T_HALF = T // 2
V_SHARD = 8192
D = 2048
C = 16
CH = T_HALF // C


def kernel(ids, E):
    def body(rc_ref, mask_ref, e_ref, out_ref,
             gbuf, ysend, yrecv, xrecv,
             gsems, osem, ysend_sems, yrecv_sems, xsend_sems, xrecv_sems):
        my_x = lax.axis_index("x")
        my_y = lax.axis_index("y")

        barrier_sem = pltpu.get_barrier_semaphore()
        pl.semaphore_signal(barrier_sem, inc=1, device_id=(my_x, 1 - my_y),
                            device_id_type=pl.DeviceIdType.MESH)
        pl.semaphore_signal(barrier_sem, inc=1, device_id=(1 - my_x, my_y),
                            device_id_type=pl.DeviceIdType.MESH)
        pl.semaphore_wait(barrier_sem, 2)

        def row_copy(i, base, slot):
            return pltpu.make_async_copy(
                e_ref.at[pl.ds(rc_ref[i], 1)],
                gbuf.at[slot, pl.ds(i - base, 1)],
                gsems.at[slot],
            )

        def issue_block(c):
            base, slot = c * CH, c % 2

            def f(i, _):
                row_copy(i, base, slot).start()
                return 0

            lax.fori_loop(base, base + CH, f, 0)

        def drain_block(c):
            base, slot = c * CH, c % 2

            def f(i, _):
                row_copy(i, base, slot).wait()
                return 0

            lax.fori_loop(base, base + CH, f, 0)

        def y_rdma(c):
            ch = pl.ds(c * CH, CH)
            return pltpu.make_async_remote_copy(
                src_ref=ysend.at[ch],
                dst_ref=yrecv.at[ch],
                send_sem=ysend_sems.at[c],
                recv_sem=yrecv_sems.at[c],
                device_id=(my_x, 1 - my_y),
                device_id_type=pl.DeviceIdType.MESH,
            )

        def x_rdma(c):
            ch = pl.ds(c * CH, CH)
            return pltpu.make_async_remote_copy(
                src_ref=ysend.at[ch],
                dst_ref=xrecv.at[ch],
                send_sem=xsend_sems.at[c],
                recv_sem=xrecv_sems.at[c],
                device_id=(1 - my_x, my_y),
                device_id_type=pl.DeviceIdType.MESH,
            )

        issue_block(0)
        for c in range(C):
            if c + 1 < C:
                issue_block(c + 1)
            drain_block(c)
            ch = pl.ds(c * CH, CH)
            ysend[ch] = (gbuf[c % 2] * mask_ref[ch]).astype(jnp.bfloat16)
            y_rdma(c).start()

        for c in range(C):
            y_rdma(c).wait()
            ch = pl.ds(c * CH, CH)
            ysend[ch] = ysend[ch] + yrecv[ch]
            x_rdma(c).start()
            pltpu.make_async_copy(
                ysend.at[ch],
                out_ref.at[pl.ds(my_x * T_HALF + c * CH, CH)],
                osem,
            ).start()

        for c in range(C):
            x_rdma(c).wait()
            ch = pl.ds(c * CH, CH)
            pltpu.make_async_copy(
                xrecv.at[ch],
                out_ref.at[pl.ds((1 - my_x) * T_HALF + c * CH, CH)],
                osem,
            ).start()

        for _ in range(2 * C):
            pltpu.make_async_copy(
                xrecv.at[pl.ds(0, CH)], out_ref.at[pl.ds(0, CH)], osem
            ).wait()

    my_x = lax.axis_index("x")
    my_y = lax.axis_index("y")
    ids_half = lax.dynamic_slice(ids, (my_x * T_HALF,), (T_HALF,))
    r = ids_half - my_y * V_SHARD
    mask = ((r >= 0) & (r < V_SHARD)).astype(jnp.float32).reshape(T_HALF, 1)
    rc = jnp.clip(r, 0, V_SHARD - 1).astype(jnp.int32)

    return pl.pallas_call(
        body,
        out_shape=jax.ShapeDtypeStruct((T, D), jnp.bfloat16),
        in_specs=[
            pl.BlockSpec(memory_space=pltpu.SMEM),
            pl.BlockSpec(memory_space=pltpu.VMEM),
            pl.BlockSpec(memory_space=pl.ANY),
        ],
        out_specs=pl.BlockSpec(memory_space=pl.ANY),
        scratch_shapes=[
            pltpu.VMEM((2, CH, D), jnp.float32),
            pltpu.VMEM((T_HALF, D), jnp.bfloat16),
            pltpu.VMEM((T_HALF, D), jnp.bfloat16),
            pltpu.VMEM((T_HALF, D), jnp.bfloat16),
            pltpu.SemaphoreType.DMA((2,)),
            pltpu.SemaphoreType.DMA,
            pltpu.SemaphoreType.DMA((C,)),
            pltpu.SemaphoreType.DMA((C,)),
            pltpu.SemaphoreType.DMA((C,)),
            pltpu.SemaphoreType.DMA((C,)),
        ],
        compiler_params=pltpu.CompilerParams(collective_id=0),
    )(rc, mask, E)


# device time: 166433 ns/iter; 1.6575x vs baseline; 1.0885x over previous
import jax
import jax.numpy as jnp
from jax import lax
from jax.experimental import pallas as pl
from jax.experimental.pallas import tpu as pltpu

T = 4096
T_HALF = T // 2
V_SHARD = 8192
D = 2048
C = 16
CH = T_HALF // C


def kernel(ids, E):
    def body(rc_ref, mask_ref, e_ref, out_ref,
             gbuf, ysend, yrecv, xrecv,
             gsems, osem, ysend_sems, yrecv_sems, xsend_sems, xrecv_sems):
        my_x = lax.axis_index("x")
        my_y = lax.axis_index("y")

        barrier_sem = pltpu.get_barrier_semaphore()
        pl.semaphore_signal(barrier_sem, inc=1, device_id=(my_x, 1 - my_y),
                            device_id_type=pl.DeviceIdType.MESH)
        pl.semaphore_signal(barrier_sem, inc=1, device_id=(1 - my_x, my_y),
                            device_id_type=pl.DeviceIdType.MESH)
        pl.semaphore_wait(barrier_sem, 2)

        def row_copy(i, base, slot):
            return pltpu.make_async_copy(
                e_ref.at[pl.ds(rc_ref[i], 1)],
                gbuf.at[slot, pl.ds(i - base, 1)],
                gsems.at[slot],
            )

        def issue_block(c):
            base, slot = c * CH, c % 2

            def f(i, _):
                @pl.when(rc_ref[i] >= 0)
                def _():
                    row_copy(i, base, slot).start()

                return 0

            lax.fori_loop(base, base + CH, f, 0)

        def drain_block(c):
            base, slot = c * CH, c % 2

            def f(i, _):
                @pl.when(rc_ref[i] >= 0)
                def _():
                    row_copy(i, base, slot).wait()

                return 0

            lax.fori_loop(base, base + CH, f, 0)

        def y_rdma(c):
            ch = pl.ds(c * CH, CH)
            return pltpu.make_async_remote_copy(
                src_ref=ysend.at[ch],
                dst_ref=yrecv.at[ch],
                send_sem=ysend_sems.at[c],
                recv_sem=yrecv_sems.at[c],
                device_id=(my_x, 1 - my_y),
                device_id_type=pl.DeviceIdType.MESH,
            )

        def x_rdma(c):
            ch = pl.ds(c * CH, CH)
            return pltpu.make_async_remote_copy(
                src_ref=ysend.at[ch],
                dst_ref=xrecv.at[ch],
                send_sem=xsend_sems.at[c],
                recv_sem=xrecv_sems.at[c],
                device_id=(1 - my_x, my_y),
                device_id_type=pl.DeviceIdType.MESH,
            )

        def gather_and_ysend(c):
            drain_block(c)
            ch = pl.ds(c * CH, CH)
            ysend[ch] = jnp.where(
                mask_ref[ch] > 0, gbuf[c % 2], 0.0
            ).astype(jnp.bfloat16)
            y_rdma(c).start()

        def reduce_and_xsend(c):
            y_rdma(c).wait()
            ch = pl.ds(c * CH, CH)
            ysend[ch] = ysend[ch] + yrecv[ch]
            x_rdma(c).start()
            pltpu.make_async_copy(
                ysend.at[ch],
                out_ref.at[pl.ds(my_x * T_HALF + c * CH, CH)],
                osem,
            ).start()

        def store_other(c):
            x_rdma(c).wait()
            ch = pl.ds(c * CH, CH)
            pltpu.make_async_copy(
                xrecv.at[ch],
                out_ref.at[pl.ds((1 - my_x) * T_HALF + c * CH, CH)],
                osem,
            ).start()

        issue_block(0)
        for c in range(C + 2):
            if c + 1 < C:
                issue_block(c + 1)
            if 1 <= c <= C:
                reduce_and_xsend(c - 1)
            if 2 <= c:
                store_other(c - 2)
            if c < C:
                gather_and_ysend(c)

        for _ in range(2 * C):
            pltpu.make_async_copy(
                xrecv.at[pl.ds(0, CH)], out_ref.at[pl.ds(0, CH)], osem
            ).wait()

    my_x = lax.axis_index("x")
    my_y = lax.axis_index("y")
    ids_half = lax.dynamic_slice(ids, (my_x * T_HALF,), (T_HALF,))
    r = ids_half - my_y * V_SHARD
    in_range = (r >= 0) & (r < V_SHARD)
    mask = in_range.astype(jnp.float32).reshape(T_HALF, 1)
    rc = jnp.where(in_range, r, -1).astype(jnp.int32)

    return pl.pallas_call(
        body,
        out_shape=jax.ShapeDtypeStruct((T, D), jnp.bfloat16),
        in_specs=[
            pl.BlockSpec(memory_space=pltpu.SMEM),
            pl.BlockSpec(memory_space=pltpu.VMEM),
            pl.BlockSpec(memory_space=pl.ANY),
        ],
        out_specs=pl.BlockSpec(memory_space=pl.ANY),
        scratch_shapes=[
            pltpu.VMEM((2, CH, D), jnp.float32),
            pltpu.VMEM((T_HALF, D), jnp.bfloat16),
            pltpu.VMEM((T_HALF, D), jnp.bfloat16),
            pltpu.VMEM((T_HALF, D), jnp.bfloat16),
            pltpu.SemaphoreType.DMA((2,)),
            pltpu.SemaphoreType.DMA,
            pltpu.SemaphoreType.DMA((C,)),
            pltpu.SemaphoreType.DMA((C,)),
            pltpu.SemaphoreType.DMA((C,)),
            pltpu.SemaphoreType.DMA((C,)),
        ],
        compiler_params=pltpu.CompilerParams(collective_id=0),
    )(rc, mask, E)


# device time: 150190 ns/iter; 1.8367x vs baseline; 1.1081x over previous
import jax
import jax.numpy as jnp
from jax import lax
from jax.experimental import pallas as pl
from jax.experimental.pallas import tpu as pltpu

T = 4096
T_HALF = T // 2
V_SHARD = 8192
D = 2048
C = 16
CH = T_HALF // C


def kernel(ids, E):
    def body(rc_ref, cnt_ref, mask_ref, e_ref, out_ref,
             gbuf, ysend, yrecv, xrecv,
             gsems, osem, ysend_sems, yrecv_sems, xsend_sems, xrecv_sems):
        my_x = lax.axis_index("x")
        my_y = lax.axis_index("y")

        barrier_sem = pltpu.get_barrier_semaphore()
        pl.semaphore_signal(barrier_sem, inc=1, device_id=(my_x, 1 - my_y),
                            device_id_type=pl.DeviceIdType.MESH)
        pl.semaphore_signal(barrier_sem, inc=1, device_id=(1 - my_x, my_y),
                            device_id_type=pl.DeviceIdType.MESH)
        pl.semaphore_wait(barrier_sem, 2)

        def row_copy(i, base, slot):
            return pltpu.make_async_copy(
                e_ref.at[pl.ds(rc_ref[i], 1)],
                gbuf.at[slot, pl.ds(i - base, 1)],
                gsems.at[slot],
            )

        def issue_block(c):
            base, slot = c * CH, c % 2

            def f(i, _):
                @pl.when(rc_ref[i] >= 0)
                def _():
                    row_copy(i, base, slot).start()

                return 0

            lax.fori_loop(base, base + CH, f, 0, unroll=4)

        def drain_block(c):
            dummy = pltpu.make_async_copy(
                e_ref.at[pl.ds(0, 1)], gbuf.at[c % 2, pl.ds(0, 1)],
                gsems.at[c % 2],
            )

            def f(i, _):
                dummy.wait()
                return 0

            lax.fori_loop(0, cnt_ref[c], f, 0)

        def y_rdma(c):
            ch = pl.ds(c * CH, CH)
            return pltpu.make_async_remote_copy(
                src_ref=ysend.at[ch],
                dst_ref=yrecv.at[ch],
                send_sem=ysend_sems.at[c],
                recv_sem=yrecv_sems.at[c],
                device_id=(my_x, 1 - my_y),
                device_id_type=pl.DeviceIdType.MESH,
            )

        def x_rdma(c):
            ch = pl.ds(c * CH, CH)
            return pltpu.make_async_remote_copy(
                src_ref=ysend.at[ch],
                dst_ref=xrecv.at[ch],
                send_sem=xsend_sems.at[c],
                recv_sem=xrecv_sems.at[c],
                device_id=(1 - my_x, my_y),
                device_id_type=pl.DeviceIdType.MESH,
            )

        def gather_and_ysend(c):
            drain_block(c)
            ch = pl.ds(c * CH, CH)
            ysend[ch] = jnp.where(
                mask_ref[ch] > 0, gbuf[c % 2], 0.0
            ).astype(jnp.bfloat16)
            y_rdma(c).start()

        def reduce_and_xsend(c):
            y_rdma(c).wait()
            ch = pl.ds(c * CH, CH)
            ysend[ch] = ysend[ch] + yrecv[ch]
            x_rdma(c).start()
            pltpu.make_async_copy(
                ysend.at[ch],
                out_ref.at[pl.ds(my_x * T_HALF + c * CH, CH)],
                osem,
            ).start()

        def store_other(c):
            x_rdma(c).wait()
            ch = pl.ds(c * CH, CH)
            pltpu.make_async_copy(
                xrecv.at[ch],
                out_ref.at[pl.ds((1 - my_x) * T_HALF + c * CH, CH)],
                osem,
            ).start()

        issue_block(0)
        for c in range(C + 2):
            if c + 1 < C:
                issue_block(c + 1)
            if 1 <= c <= C:
                reduce_and_xsend(c - 1)
            if 2 <= c:
                store_other(c - 2)
            if c < C:
                gather_and_ysend(c)

        for _ in range(2 * C):
            pltpu.make_async_copy(
                xrecv.at[pl.ds(0, CH)], out_ref.at[pl.ds(0, CH)], osem
            ).wait()

    my_x = lax.axis_index("x")
    my_y = lax.axis_index("y")
    ids_half = lax.dynamic_slice(ids, (my_x * T_HALF,), (T_HALF,))
    r = ids_half - my_y * V_SHARD
    in_range = (r >= 0) & (r < V_SHARD)
    mask = in_range.astype(jnp.float32).reshape(T_HALF, 1)
    rc = jnp.where(in_range, r, -1).astype(jnp.int32)
    cnts = in_range.reshape(C, CH).sum(axis=1).astype(jnp.int32)

    return pl.pallas_call(
        body,
        out_shape=jax.ShapeDtypeStruct((T, D), jnp.bfloat16),
        in_specs=[
            pl.BlockSpec(memory_space=pltpu.SMEM),
            pl.BlockSpec(memory_space=pltpu.SMEM),
            pl.BlockSpec(memory_space=pltpu.VMEM),
            pl.BlockSpec(memory_space=pl.ANY),
        ],
        out_specs=pl.BlockSpec(memory_space=pl.ANY),
        scratch_shapes=[
            pltpu.VMEM((2, CH, D), jnp.float32),
            pltpu.VMEM((T_HALF, D), jnp.bfloat16),
            pltpu.VMEM((T_HALF, D), jnp.bfloat16),
            pltpu.VMEM((T_HALF, D), jnp.bfloat16),
            pltpu.SemaphoreType.DMA((2,)),
            pltpu.SemaphoreType.DMA,
            pltpu.SemaphoreType.DMA((C,)),
            pltpu.SemaphoreType.DMA((C,)),
            pltpu.SemaphoreType.DMA((C,)),
            pltpu.SemaphoreType.DMA((C,)),
        ],
        compiler_params=pltpu.CompilerParams(collective_id=0),
    )(rc, cnts, mask, E)


# device time: 115818 ns/iter; 2.3818x vs baseline; 1.2968x over previous
import jax
import jax.numpy as jnp
from jax import lax
from jax.experimental import pallas as pl
from jax.experimental.pallas import tpu as pltpu

T = 4096
T_HALF = T // 2
V_SHARD = 8192
D = 2048
C = 16
CH = T_HALF // C


def kernel(ids, E):
    def body(rc_ref, cnt_ref, mask_ref, e_ref, out_ref,
             gbuf, ysend, yrecv, xrecv,
             gsems, osem, ysend_sems, yrecv_sems, xsend_sems, xrecv_sems):
        my_x = lax.axis_index("x")
        my_y = lax.axis_index("y")

        barrier_sem = pltpu.get_barrier_semaphore()
        pl.semaphore_signal(barrier_sem, inc=1, device_id=(my_x, 1 - my_y),
                            device_id_type=pl.DeviceIdType.MESH)
        pl.semaphore_signal(barrier_sem, inc=1, device_id=(1 - my_x, my_y),
                            device_id_type=pl.DeviceIdType.MESH)
        pl.semaphore_wait(barrier_sem, 2)

        def row_copy(i, base, slot):
            return pltpu.make_async_copy(
                e_ref.at[pl.ds(rc_ref[i], 1)],
                gbuf.at[slot, pl.ds(i - base, 1)],
                gsems.at[slot],
            )

        def issue_block(c):
            base, slot = c * CH, c % 2

            def f(i, _):
                @pl.when(rc_ref[i] >= 0)
                def _():
                    row_copy(i, base, slot).start()

                return 0

            lax.fori_loop(base, base + CH, f, 0, unroll=4)

        def drain_block(c):
            dummy = pltpu.make_async_copy(
                e_ref.at[pl.ds(0, 1)], gbuf.at[c % 2, pl.ds(0, 1)],
                gsems.at[c % 2],
            )

            def f(i, _):
                dummy.wait()
                return 0

            lax.fori_loop(0, cnt_ref[c], f, 0)

        def y_rdma(c):
            ch = pl.ds(c * CH, CH)
            return pltpu.make_async_remote_copy(
                src_ref=ysend.at[ch],
                dst_ref=yrecv.at[ch],
                send_sem=ysend_sems.at[c],
                recv_sem=yrecv_sems.at[c],
                device_id=(my_x, 1 - my_y),
                device_id_type=pl.DeviceIdType.MESH,
            )

        def x_rdma(c):
            ch = pl.ds(c * CH, CH)
            return pltpu.make_async_remote_copy(
                src_ref=ysend.at[ch],
                dst_ref=xrecv.at[ch],
                send_sem=xsend_sems.at[c],
                recv_sem=xrecv_sems.at[c],
                device_id=(1 - my_x, my_y),
                device_id_type=pl.DeviceIdType.MESH,
            )

        def gather_and_ysend(c):
            drain_block(c)
            ch = pl.ds(c * CH, CH)
            ysend[ch] = jnp.where(
                mask_ref[ch] > 0, gbuf[c % 2], 0.0
            ).astype(jnp.bfloat16)
            y_rdma(c).start()

        def reduce_and_xsend(c):
            y_rdma(c).wait()
            ch = pl.ds(c * CH, CH)
            ysend[ch] = ysend[ch] + yrecv[ch]
            x_rdma(c).start()
            pltpu.make_async_copy(
                ysend.at[ch],
                out_ref.at[pl.ds(my_x * T_HALF + c * CH, CH)],
                osem,
            ).start()

        def store_other(c):
            x_rdma(c).wait()
            ch = pl.ds(c * CH, CH)
            pltpu.make_async_copy(
                xrecv.at[ch],
                out_ref.at[pl.ds((1 - my_x) * T_HALF + c * CH, CH)],
                osem,
            ).start()

        L1, L2 = 2, 4
        issue_block(0)
        for c in range(C + L2):
            if c + 1 < C:
                issue_block(c + 1)
            if c < C:
                gather_and_ysend(c)
            if L1 <= c < C + L1:
                reduce_and_xsend(c - L1)
            if L2 <= c:
                store_other(c - L2)

        for _ in range(2 * C):
            pltpu.make_async_copy(
                xrecv.at[pl.ds(0, CH)], out_ref.at[pl.ds(0, CH)], osem
            ).wait()

    my_x = lax.axis_index("x")
    my_y = lax.axis_index("y")
    ids_half = lax.dynamic_slice(ids, (my_x * T_HALF,), (T_HALF,))
    r = ids_half - my_y * V_SHARD
    in_range = (r >= 0) & (r < V_SHARD)
    mask = in_range.astype(jnp.float32).reshape(T_HALF, 1)
    rc = jnp.where(in_range, r, -1).astype(jnp.int32)
    cnts = in_range.reshape(C, CH).sum(axis=1).astype(jnp.int32)

    return pl.pallas_call(
        body,
        out_shape=jax.ShapeDtypeStruct((T, D), jnp.bfloat16),
        in_specs=[
            pl.BlockSpec(memory_space=pltpu.SMEM),
            pl.BlockSpec(memory_space=pltpu.SMEM),
            pl.BlockSpec(memory_space=pltpu.VMEM),
            pl.BlockSpec(memory_space=pl.ANY),
        ],
        out_specs=pl.BlockSpec(memory_space=pl.ANY),
        scratch_shapes=[
            pltpu.VMEM((2, CH, D), jnp.float32),
            pltpu.VMEM((T_HALF, D), jnp.bfloat16),
            pltpu.VMEM((T_HALF, D), jnp.bfloat16),
            pltpu.VMEM((T_HALF, D), jnp.bfloat16),
            pltpu.SemaphoreType.DMA((2,)),
            pltpu.SemaphoreType.DMA,
            pltpu.SemaphoreType.DMA((C,)),
            pltpu.SemaphoreType.DMA((C,)),
            pltpu.SemaphoreType.DMA((C,)),
            pltpu.SemaphoreType.DMA((C,)),
        ],
        compiler_params=pltpu.CompilerParams(collective_id=0),
    )(rc, cnts, mask, E)
